# Initial kernel scaffold; baseline (speedup 1.0000x reference)
import functools

import jax
import jax.numpy as jnp
from jax import lax
from jax.experimental import pallas as pl
from jax.experimental.pallas import tpu as pltpu
from jax.experimental.pallas import tpu_sc as plsc

L = 16
N = 256
NCHUNK = N // L
NC = 2
B = 16


def _argmax16(load_chunk):
    lane = lax.iota(jnp.int32, L)
    vmax = load_chunk(0)
    vchunk = jnp.zeros((L,), jnp.int32)
    for j in range(1, NCHUNK):
        v = load_chunk(j)
        upd = v > vmax
        vmax = jnp.where(upd, v, vmax)
        vchunk = jnp.where(upd, jnp.int32(j), vchunk)
    m = jnp.max(vmax)
    cand = jnp.where(vmax == m, vchunk * L + lane, jnp.int32(1 << 30))
    idx = jnp.min(cand)
    return m, idx


def _make_kernel():
    mesh = plsc.VectorSubcoreMesh(core_axis_name="c", subcore_axis_name="s")

    @functools.partial(
        pl.kernel,
        mesh=mesh,
        out_type=jax.ShapeDtypeStruct((B, N, N), jnp.float32),
        scratch_types=[
            pltpu.VMEM((N, N), jnp.float32),
            pltpu.VMEM((N,), jnp.float32),
            pltpu.VMEM((N,), jnp.int32),
            pltpu.VMEM((N,), jnp.int32),
        ],
    )
    def greedy(s_hbm, out_hbm, st, cached, picks_r, picks_c):
        wid = lax.axis_index("s") * NC + lax.axis_index("c")

        @pl.when(wid < B)
        def _body():
            lane = lax.iota(jnp.int32, L)
            lane0 = lane == 0
            zero = jnp.zeros((L,), jnp.float32)

            pltpu.sync_copy(s_hbm.at[wid], st)

            def set1f(ref, i, val):
                plsc.store_scatter(ref, [jnp.broadcast_to(i, (L,))],
                                   jnp.broadcast_to(val, (L,)), mask=lane0)

            def init_row(rr, carry):
                vmax = st[rr, pl.ds(0, L)]
                for j in range(1, NCHUNK):
                    vmax = jnp.maximum(vmax, st[rr, pl.ds(j * L, L)])
                set1f(cached, rr, jnp.max(vmax))
                return carry
            lax.fori_loop(0, N, init_row, 0)

            def argmax_cached():
                return _argmax16(lambda j: cached[pl.ds(j * L, L)])

            def rescan(rr):
                return _argmax16(lambda j: st[rr, pl.ds(j * L, L)])

            def step(i, carry):
                m, r = argmax_cached()
                tm, c = rescan(r)

                def stale(cy):
                    return cy[0] != cy[1]

                def fix(cy):
                    m_, tm_, r_, _ = cy
                    set1f(cached, r_, tm_)
                    m2, r2 = argmax_cached()
                    tm2, c2 = rescan(r2)
                    return (m2, tm2, r2, c2)

                m, tm, r, c = lax.while_loop(stale, fix, (m, tm, r, c))

                for j in range(NCHUNK):
                    st[r, pl.ds(j * L, L)] = zero
                set1f(cached, r, jnp.float32(0.0))
                cvec = jnp.broadcast_to(c, (L,))
                for j in range(NCHUNK):
                    plsc.store_scatter(st, [lane + jnp.int32(j * L), cvec],
                                       zero)
                plsc.store_scatter(picks_r, [jnp.broadcast_to(i, (L,))],
                                   jnp.broadcast_to(r, (L,)), mask=lane0)
                plsc.store_scatter(picks_c, [jnp.broadcast_to(i, (L,))],
                                   jnp.broadcast_to(c, (L,)), mask=lane0)
                return carry
            lax.fori_loop(0, N, step, 0)

            def clear_row(rr, carry):
                for j in range(NCHUNK):
                    st[rr, pl.ds(j * L, L)] = zero
                return carry
            lax.fori_loop(0, N, clear_row, 0)

            ones = jnp.ones((L,), jnp.float32)
            for j in range(NCHUNK):
                plsc.store_scatter(
                    st, [picks_r[pl.ds(j * L, L)], picks_c[pl.ds(j * L, L)]],
                    ones)

            pltpu.sync_copy(st, out_hbm.at[wid])

    return greedy


_greedy_kernel = _make_kernel()


@jax.jit
def kernel(s):
    return _greedy_kernel(s)


# speedup vs baseline: 51.1352x; 51.1352x over previous
import functools

import jax
import jax.numpy as jnp
from jax import lax
from jax.experimental import pallas as pl
from jax.experimental.pallas import tpu as pltpu
from jax.experimental.pallas import tpu_sc as plsc

L = 16
N = 256
NCHUNK = N // L
NC = 2
B = 16


def _argmax16(load_chunk):
    lane = lax.iota(jnp.int32, L)
    vmax = load_chunk(0)
    vchunk = jnp.zeros((L,), jnp.int32)
    for j in range(1, NCHUNK):
        v = load_chunk(j)
        upd = v > vmax
        vmax = jnp.where(upd, v, vmax)
        vchunk = jnp.where(upd, jnp.int32(j), vchunk)
    m = jnp.max(vmax)
    cand = jnp.where(vmax == m, vchunk * L + lane, jnp.int32(1 << 30))
    idx = jnp.min(cand)
    return m, idx


def _make_kernel():
    mesh = plsc.VectorSubcoreMesh(core_axis_name="c", subcore_axis_name="s")

    @functools.partial(
        pl.kernel,
        mesh=mesh,
        out_type=jax.ShapeDtypeStruct((B, N, N), jnp.float32),
        compiler_params=pltpu.CompilerParams(needs_layout_passes=False),
        scratch_types=[
            pltpu.VMEM((N, N), jnp.float32),
            pltpu.VMEM((N,), jnp.float32),
            pltpu.VMEM((N,), jnp.int32),
            pltpu.VMEM((N,), jnp.int32),
        ],
    )
    def greedy(s_hbm, out_hbm, st, cached, picks_r, picks_c):
        wid = lax.axis_index("s") * NC + lax.axis_index("c")

        @pl.when(wid < B)
        def _body():
            lane = lax.iota(jnp.int32, L)
            lane0 = lane == 0
            zero = jnp.zeros((L,), jnp.float32)

            pltpu.sync_copy(s_hbm.at[wid], st)

            def set1f(ref, i, val):
                plsc.store_scatter(ref, [jnp.broadcast_to(i, (L,))],
                                   jnp.broadcast_to(val, (L,)), mask=lane0)

            def init_row(rr, carry):
                vmax = st[rr, pl.ds(0, L)]
                for j in range(1, NCHUNK):
                    vmax = jnp.maximum(vmax, st[rr, pl.ds(j * L, L)])
                set1f(cached, rr, jnp.max(vmax))
                return carry
            lax.fori_loop(0, N, init_row, 0)

            def argmax_cached():
                return _argmax16(lambda j: cached[pl.ds(j * L, L)])

            def rescan(rr):
                return _argmax16(lambda j: st[rr, pl.ds(j * L, L)])

            def step(i, carry):
                m, r = argmax_cached()
                tm, c = rescan(r)

                def stale(cy):
                    return cy[0] != cy[1]

                def fix(cy):
                    m_, tm_, r_, _ = cy
                    set1f(cached, r_, tm_)
                    m2, r2 = argmax_cached()
                    tm2, c2 = rescan(r2)
                    return (m2, tm2, r2, c2)

                m, tm, r, c = lax.while_loop(stale, fix, (m, tm, r, c))

                for j in range(NCHUNK):
                    st[r, pl.ds(j * L, L)] = zero
                set1f(cached, r, jnp.float32(0.0))
                cvec = jnp.broadcast_to(c, (L,))
                for j in range(NCHUNK):
                    plsc.store_scatter(st, [lane + jnp.int32(j * L), cvec],
                                       zero)
                plsc.store_scatter(picks_r, [jnp.broadcast_to(i, (L,))],
                                   jnp.broadcast_to(r, (L,)), mask=lane0)
                plsc.store_scatter(picks_c, [jnp.broadcast_to(i, (L,))],
                                   jnp.broadcast_to(c, (L,)), mask=lane0)
                return carry
            lax.fori_loop(0, N, step, 0)

            def clear_row(rr, carry):
                for j in range(NCHUNK):
                    st[rr, pl.ds(j * L, L)] = zero
                return carry
            lax.fori_loop(0, N, clear_row, 0)

            ones = jnp.ones((L,), jnp.float32)
            for j in range(NCHUNK):
                plsc.store_scatter(
                    st, [picks_r[pl.ds(j * L, L)], picks_c[pl.ds(j * L, L)]],
                    ones)

            pltpu.sync_copy(st, out_hbm.at[wid])

    return greedy


_greedy_kernel = _make_kernel()


@jax.jit
def kernel(s):
    return _greedy_kernel(s)


# speedup vs baseline: 78.3124x; 1.5315x over previous
import functools

import jax
import jax.numpy as jnp
from jax import lax
from jax.experimental import pallas as pl
from jax.experimental.pallas import tpu as pltpu
from jax.experimental.pallas import tpu_sc as plsc

L = 16
N = 256
NCHUNK = N // L
NC = 2
B = 16
BIG = jnp.int32(1 << 30)


def _argmax16(load_chunk):
    lane = lax.iota(jnp.int32, L)
    pairs = [(load_chunk(j), jnp.full((L,), jnp.int32(j * L)))
             for j in range(NCHUNK)]
    while len(pairs) > 1:
        nxt = []
        for k in range(0, len(pairs), 2):
            (va, ia), (vb, ib) = pairs[k], pairs[k + 1]
            take_b = vb > va
            nxt.append((jnp.where(take_b, vb, va),
                        jnp.where(take_b, ib, ia)))
        pairs = nxt
    vmax, vbase = pairs[0]
    m = jnp.max(vmax)
    cand = jnp.where(vmax == m, vbase + lane, BIG)
    idx = jnp.min(cand)
    return m, idx


def _make_kernel():
    mesh = plsc.VectorSubcoreMesh(core_axis_name="c", subcore_axis_name="s")

    @functools.partial(
        pl.kernel,
        mesh=mesh,
        out_type=jax.ShapeDtypeStruct((B, N, N), jnp.float32),
        compiler_params=pltpu.CompilerParams(needs_layout_passes=False),
        scratch_types=[
            pltpu.VMEM((N, N), jnp.float32),
            pltpu.VMEM((N,), jnp.float32),
            pltpu.VMEM((N,), jnp.float32),
            pltpu.VMEM((N,), jnp.int32),
            pltpu.VMEM((N,), jnp.int32),
        ],
    )
    def greedy(s_hbm, out_hbm, st, cached, colmask, picks_r, picks_c):
        wid = lax.axis_index("s") * NC + lax.axis_index("c")

        @pl.when(wid < B)
        def _body():
            lane = lax.iota(jnp.int32, L)
            lane0 = lane == 0
            zero = jnp.zeros((L,), jnp.float32)
            ones = jnp.ones((L,), jnp.float32)

            pltpu.sync_copy(s_hbm.at[wid], st)

            def set1(ref, i, val):
                plsc.store_scatter(ref, [jnp.broadcast_to(i, (L,))],
                                   jnp.broadcast_to(val, (L,)), mask=lane0)

            for j in range(NCHUNK):
                colmask[pl.ds(j * L, L)] = ones

            def init_row(rr, carry):
                vmax = st[rr, pl.ds(0, L)]
                for j in range(1, NCHUNK):
                    vmax = jnp.maximum(vmax, st[rr, pl.ds(j * L, L)])
                set1(cached, rr, jnp.max(vmax))
                return carry
            lax.fori_loop(0, N, init_row, 0)

            def argmax_cached():
                return _argmax16(lambda j: cached[pl.ds(j * L, L)])

            def rescan(rr):
                return _argmax16(
                    lambda j: st[rr, pl.ds(j * L, L)]
                    * colmask[pl.ds(j * L, L)])

            def step(i, carry):
                m, r = argmax_cached()
                tm, c = rescan(r)

                def stale(cy):
                    return cy[0] != cy[1]

                def fix(cy):
                    m_, tm_, r_, _ = cy
                    set1(cached, r_, tm_)
                    m2, r2 = argmax_cached()
                    tm2, c2 = rescan(r2)
                    return (m2, tm2, r2, c2)

                m, tm, r, c = lax.while_loop(stale, fix, (m, tm, r, c))

                for j in range(NCHUNK):
                    st[r, pl.ds(j * L, L)] = zero
                set1(cached, r, jnp.float32(0.0))
                set1(colmask, c, jnp.float32(0.0))
                set1(picks_r, i, r)
                set1(picks_c, i, c)
                return carry
            lax.fori_loop(0, N, step, 0)

            def clear_row(rr, carry):
                for j in range(NCHUNK):
                    st[rr, pl.ds(j * L, L)] = zero
                return carry
            lax.fori_loop(0, N, clear_row, 0)

            for j in range(NCHUNK):
                plsc.store_scatter(
                    st, [picks_r[pl.ds(j * L, L)], picks_c[pl.ds(j * L, L)]],
                    ones)

            pltpu.sync_copy(st, out_hbm.at[wid])

    return greedy


_greedy_kernel = _make_kernel()


@jax.jit
def kernel(s):
    return _greedy_kernel(s)


# speedup vs baseline: 79.2449x; 1.0119x over previous
import functools

import jax
import jax.numpy as jnp
from jax import lax
from jax.experimental import pallas as pl
from jax.experimental.pallas import tpu as pltpu
from jax.experimental.pallas import tpu_sc as plsc

L = 16
N = 256
NCHUNK = N // L
NC = 2
B = 16
BIG = 1 << 30


def _argmax16(load_chunk):
    lane = lax.iota(jnp.int32, L)
    pairs = [(load_chunk(j), jnp.full((L,), jnp.int32(j * L)))
             for j in range(NCHUNK)]
    while len(pairs) > 1:
        nxt = []
        for k in range(0, len(pairs), 2):
            (va, ia), (vb, ib) = pairs[k], pairs[k + 1]
            take_b = vb > va
            nxt.append((jnp.where(take_b, vb, va),
                        jnp.where(take_b, ib, ia)))
        pairs = nxt
    vmax, vbase = pairs[0]
    m = jnp.max(vmax)
    cand = jnp.where(vmax == m, vbase + lane, BIG)
    idx = jnp.min(cand)
    return m, idx


def _make_kernel():
    mesh = plsc.VectorSubcoreMesh(core_axis_name="c", subcore_axis_name="s")

    @functools.partial(
        pl.kernel,
        mesh=mesh,
        out_type=jax.ShapeDtypeStruct((B, N, N), jnp.float32),
        compiler_params=pltpu.CompilerParams(needs_layout_passes=False),
        scratch_types=[
            pltpu.VMEM((N, N), jnp.float32),
            pltpu.VMEM((N,), jnp.float32),
            pltpu.VMEM((N,), jnp.int32),
            pltpu.VMEM((N,), jnp.int32),
        ],
    )
    def greedy(s_hbm, out_hbm, st, cached, picks_r, picks_c):
        wid = lax.axis_index("s") * NC + lax.axis_index("c")

        @pl.when(wid < B)
        def _body():
            lane = lax.iota(jnp.int32, L)
            lane0 = lane == 0
            zero = jnp.zeros((L,), jnp.float32)
            ones = jnp.ones((L,), jnp.float32)

            pltpu.sync_copy(s_hbm.at[wid], st)

            def set1(ref, i, val):
                plsc.store_scatter(ref, [jnp.broadcast_to(i, (L,))],
                                   jnp.broadcast_to(val, (L,)), mask=lane0)

            def init_row(rr, carry):
                vmax = st[rr, pl.ds(0, L)]
                for j in range(1, NCHUNK):
                    vmax = jnp.maximum(vmax, st[rr, pl.ds(j * L, L)])
                set1(cached, rr, jnp.max(vmax))
                return carry
            lax.fori_loop(0, N, init_row, 0)

            def argmax_cached():
                return _argmax16(lambda j: cached[pl.ds(j * L, L)])

            def step(i, cm):
                def rescan(rr):
                    return _argmax16(
                        lambda j: st[rr, pl.ds(j * L, L)] * cm[j])

                m, r = argmax_cached()
                tm, c = rescan(r)

                def stale(cy):
                    return cy[0] != cy[1]

                def fix(cy):
                    m_, tm_, r_, _ = cy
                    set1(cached, r_, tm_)
                    m2, r2 = argmax_cached()
                    tm2, c2 = rescan(r2)
                    return (m2, tm2, r2, c2)

                m, tm, r, c = lax.while_loop(stale, fix, (m, tm, r, c))

                for j in range(NCHUNK):
                    st[r, pl.ds(j * L, L)] = zero
                set1(cached, r, jnp.float32(0.0))
                set1(picks_r, i, r)
                set1(picks_c, i, c)
                cs = jnp.broadcast_to(c, (L,))
                cm = tuple(
                    jnp.where(lane + jnp.int32(j * L) == cs, 0.0, cm[j])
                    for j in range(NCHUNK))
                return cm
            lax.fori_loop(0, N, step, tuple(ones for _ in range(NCHUNK)))

            def clear_row(rr, carry):
                for j in range(NCHUNK):
                    st[rr, pl.ds(j * L, L)] = zero
                return carry
            lax.fori_loop(0, N, clear_row, 0)

            for j in range(NCHUNK):
                plsc.store_scatter(
                    st, [picks_r[pl.ds(j * L, L)], picks_c[pl.ds(j * L, L)]],
                    ones)

            pltpu.sync_copy(st, out_hbm.at[wid])

    return greedy


_greedy_kernel = _make_kernel()


@jax.jit
def kernel(s):
    return _greedy_kernel(s)
